# baseline (speedup 1.0000x reference)
import functools

import jax
import jax.numpy as jnp
from jax import lax
from jax.experimental import pallas as pl
from jax.experimental.pallas import tpu as pltpu
from jax.experimental.pallas import tpu_sc as plsc

NUM_CODES = 1024
DIM = 64
TOK = 1024
N_STEPS = 16
N_TOK = N_STEPS * TOK
N_ELEM = N_TOK * DIM
COMMIT = 0.25
EPS = 1e-10


def _vq_body(x_ref, w_ref, qst_ref, idx_ref, scal_ref, wt_s, wb_s, w2_s,
             ci_s, sse_s):
    i = pl.program_id(0)

    @pl.when(i == 0)
    def _init():
        w = w_ref[...]
        wt_s[...] = w.T.astype(jnp.bfloat16)
        wb_s[...] = -2.0 * w.astype(jnp.bfloat16)
        w2_s[...] = jnp.sum(w * w, axis=1, keepdims=True)
        ci_s[...] = lax.broadcasted_iota(jnp.int32, (NUM_CODES, 1), 0
                                         ).astype(jnp.float32)
        sse_s[...] = jnp.zeros_like(sse_s)

    xb = x_ref[0]
    x2 = jnp.sum(xb * xb, axis=0, keepdims=True)
    mmn = jnp.dot(wb_s[...], xb.astype(jnp.bfloat16),
                  preferred_element_type=jnp.float32)
    d = (x2 + w2_s[...]) + mmn
    dmin = jnp.min(d, axis=0, keepdims=True)
    cif = ci_s[...]
    idxf = jnp.min(jnp.where(d == dmin, cif, 2048.0),
                   axis=0, keepdims=True)
    idx_ref[0] = idxf.astype(jnp.int32)
    hit = (cif == idxf).astype(jnp.bfloat16)
    qt = jnp.dot(wt_s[...], hit,
                 preferred_element_type=jnp.float32)
    e = qt - xb
    qst_ref[0] = xb + e
    sse_s[...] += jnp.sum(e * e, axis=(0, 1), keepdims=True)

    @pl.when(i == N_STEPS - 1)
    def _fin():
        loss = (1.0 + COMMIT) * sse_s[...] / N_ELEM
        scal_ref[...] = jnp.broadcast_to(loss, (1, 128))


_SC_CORES = 2
_SC_WORKERS = 32
_BPW = N_TOK // _SC_WORKERS
_LANES = 16


@functools.partial(
    pl.kernel,
    mesh=plsc.VectorSubcoreMesh(core_axis_name="c", subcore_axis_name="s"),
    out_type=jax.ShapeDtypeStruct((_SC_WORKERS, NUM_CODES), jnp.float32),
    scratch_types=[
        pltpu.VMEM((_BPW,), jnp.int32),
        pltpu.VMEM((NUM_CODES,), jnp.float32),
    ],
    compiler_params=pltpu.CompilerParams(needs_layout_passes=False),
)
def _sc_hist(idx_hbm, cnt_hbm, idx_v, cnt_v):
    wid = lax.axis_index("s") * _SC_CORES + lax.axis_index("c")
    pltpu.sync_copy(idx_hbm.at[pl.ds(wid * _BPW, _BPW)], idx_v)
    for j in range(NUM_CODES // _LANES):
        cnt_v[pl.ds(j * _LANES, _LANES)] = jnp.zeros((_LANES,), jnp.float32)
    ones = jnp.full((_LANES,), 1.0, jnp.float32)
    for j in range(_BPW // _LANES):
        iv = idx_v[pl.ds(j * _LANES, _LANES)]
        plsc.addupdate_scatter(cnt_v, [iv], ones)
    pltpu.sync_copy(cnt_v, cnt_hbm.at[wid])


def _perp_body(cnt_ref, out_ref):
    cnts = jnp.sum(cnt_ref[...], axis=0, keepdims=True)
    p = cnts * (1.0 / N_TOK)
    s = jnp.sum(p * jnp.log(p + EPS), axis=1, keepdims=True)
    out_ref[...] = jnp.broadcast_to(jnp.exp(-s), (1, 128))


def kernel(x, embedding_weight):
    B, C, H, W = x.shape
    xr = x.reshape(B, C, H * W)
    qst, idx, scal = pl.pallas_call(
        _vq_body,
        grid=(N_STEPS,),
        in_specs=[
            pl.BlockSpec((1, DIM, TOK), lambda i: (i, 0, 0)),
            pl.BlockSpec((NUM_CODES, DIM), lambda i: (0, 0)),
        ],
        out_specs=[
            pl.BlockSpec((1, DIM, TOK), lambda i: (i, 0, 0)),
            pl.BlockSpec((1, 1, TOK), lambda i: (i, 0, 0)),
            pl.BlockSpec((1, 128), lambda i: (0, 0)),
        ],
        out_shape=[
            jax.ShapeDtypeStruct((B, DIM, TOK), jnp.float32),
            jax.ShapeDtypeStruct((B, 1, TOK), jnp.int32),
            jax.ShapeDtypeStruct((1, 128), jnp.float32),
        ],
        scratch_shapes=[
            pltpu.VMEM((DIM, NUM_CODES), jnp.bfloat16),
            pltpu.VMEM((NUM_CODES, DIM), jnp.bfloat16),
            pltpu.VMEM((NUM_CODES, 1), jnp.float32),
            pltpu.VMEM((NUM_CODES, 1), jnp.float32),
            pltpu.VMEM((1, 1), jnp.float32),
        ],
    )(xr, embedding_weight)
    quantized_st = qst.reshape(B, C, H, W)
    idx_out = idx.reshape(B, H, W)
    cnt32 = _sc_hist(idx.reshape(N_TOK))
    perp = pl.pallas_call(
        _perp_body,
        out_shape=jax.ShapeDtypeStruct((1, 128), jnp.float32),
    )(cnt32)
    loss = scal[0, 0]
    perplexity = perp[0, 0]
    return (quantized_st, loss, perplexity, idx_out)


# speedup vs baseline: 1.0322x; 1.0322x over previous
import functools

import jax
import jax.numpy as jnp
from jax import lax
from jax.experimental import pallas as pl
from jax.experimental.pallas import tpu as pltpu
from jax.experimental.pallas import tpu_sc as plsc

NUM_CODES = 1024
DIM = 64
TOK = 1024
N_STEPS = 16
N_TOK = N_STEPS * TOK
N_ELEM = N_TOK * DIM
COMMIT = 0.25
EPS = 1e-10


def _vq_body(x_ref, w_ref, qst_ref, idx_ref, scal_ref, wt_s, wb_s, w2_s,
             ci_s, cnt_s, sse_s):
    i = pl.program_id(0)

    @pl.when(i == 0)
    def _init():
        w = w_ref[...]
        wt_s[...] = w.T.astype(jnp.bfloat16)
        wb_s[...] = -2.0 * w.astype(jnp.bfloat16)
        w2_s[...] = jnp.sum(w * w, axis=1, keepdims=True)
        ci_s[...] = lax.broadcasted_iota(jnp.int32, (NUM_CODES, 1), 0
                                         ).astype(jnp.float32)
        cnt_s[...] = jnp.zeros_like(cnt_s)
        sse_s[...] = jnp.zeros_like(sse_s)

    xb = x_ref[0]
    x2 = jnp.sum(xb * xb, axis=0, keepdims=True)
    mmn = jnp.dot(wb_s[...], xb.astype(jnp.bfloat16),
                  preferred_element_type=jnp.float32)
    d = (x2 + w2_s[...]) + mmn
    dmin = jnp.min(d, axis=0, keepdims=True)
    cif = ci_s[...]
    idxf = jnp.min(jnp.where(d == dmin, cif, 2048.0),
                   axis=0, keepdims=True)
    idx_ref[0] = idxf.astype(jnp.int32)
    hit = (cif == idxf).astype(jnp.bfloat16)
    qt = jnp.dot(wt_s[...], hit,
                 preferred_element_type=jnp.float32)
    e = qt - xb
    qst_ref[0] = (xb + e).reshape(DIM, 32, 32)
    cnt_s[...] += jnp.sum(hit.astype(jnp.float32),
                          axis=1, keepdims=True)
    sse_s[...] += jnp.sum(e * e, axis=(0, 1), keepdims=True)

    @pl.when(i == N_STEPS - 1)
    def _fin():
        loss = (1.0 + COMMIT) * sse_s[...] / N_ELEM
        p = cnt_s[...] * (1.0 / N_TOK)
        s = jnp.sum(p * jnp.log(p + EPS), axis=0, keepdims=True)
        perp = jnp.exp(-s)
        lane = lax.broadcasted_iota(jnp.int32, (1, 128), 1)
        scal_ref[...] = jnp.where(lane == 0,
                                  jnp.broadcast_to(loss, (1, 128)),
                                  jnp.broadcast_to(perp, (1, 128)))


_SC_CORES = 2
_SC_WORKERS = 32
_BPW = N_TOK // _SC_WORKERS
_LANES = 16


@functools.partial(
    pl.kernel,
    mesh=plsc.VectorSubcoreMesh(core_axis_name="c", subcore_axis_name="s"),
    out_type=jax.ShapeDtypeStruct((_SC_WORKERS, NUM_CODES), jnp.float32),
    scratch_types=[
        pltpu.VMEM((_BPW,), jnp.int32),
        pltpu.VMEM((NUM_CODES,), jnp.float32),
    ],
    compiler_params=pltpu.CompilerParams(needs_layout_passes=False),
)
def _sc_hist(idx_hbm, cnt_hbm, idx_v, cnt_v):
    wid = lax.axis_index("s") * _SC_CORES + lax.axis_index("c")
    pltpu.sync_copy(idx_hbm.at[pl.ds(wid * _BPW, _BPW)], idx_v)
    for j in range(NUM_CODES // _LANES):
        cnt_v[pl.ds(j * _LANES, _LANES)] = jnp.zeros((_LANES,), jnp.float32)
    ones = jnp.full((_LANES,), 1.0, jnp.float32)
    for j in range(_BPW // _LANES):
        iv = idx_v[pl.ds(j * _LANES, _LANES)]
        plsc.addupdate_scatter(cnt_v, [iv], ones)
    pltpu.sync_copy(cnt_v, cnt_hbm.at[wid])


def _perp_body(cnt_ref, out_ref):
    cnts = jnp.sum(cnt_ref[...], axis=0, keepdims=True)
    p = cnts * (1.0 / N_TOK)
    s = jnp.sum(p * jnp.log(p + EPS), axis=1, keepdims=True)
    out_ref[...] = jnp.broadcast_to(jnp.exp(-s), (1, 128))


def kernel(x, embedding_weight):
    B, C, H, W = x.shape
    xr = x.reshape(B, C, H * W)
    qst, idx, scal = pl.pallas_call(
        _vq_body,
        grid=(N_STEPS,),
        in_specs=[
            pl.BlockSpec((1, DIM, TOK), lambda i: (i, 0, 0)),
            pl.BlockSpec((NUM_CODES, DIM), lambda i: (0, 0)),
        ],
        out_specs=[
            pl.BlockSpec((1, DIM, 32, 32), lambda i: (i, 0, 0, 0)),
            pl.BlockSpec((1, 1, TOK), lambda i: (i, 0, 0)),
            pl.BlockSpec((1, 128), lambda i: (0, 0)),
        ],
        out_shape=[
            jax.ShapeDtypeStruct((B, DIM, 32, 32), jnp.float32),
            jax.ShapeDtypeStruct((B, 1, TOK), jnp.int32),
            jax.ShapeDtypeStruct((1, 128), jnp.float32),
        ],
        scratch_shapes=[
            pltpu.VMEM((DIM, NUM_CODES), jnp.bfloat16),
            pltpu.VMEM((NUM_CODES, DIM), jnp.bfloat16),
            pltpu.VMEM((NUM_CODES, 1), jnp.float32),
            pltpu.VMEM((NUM_CODES, 1), jnp.float32),
            pltpu.VMEM((NUM_CODES, 1), jnp.float32),
            pltpu.VMEM((1, 1), jnp.float32),
        ],
    )(xr, embedding_weight)
    quantized_st = qst
    idx_out = idx.reshape(B, H, W)
    loss = scal[0, 0]
    perplexity = scal[0, 1]
    return (quantized_st, loss, perplexity, idx_out)


# speedup vs baseline: 1.2456x; 1.2068x over previous
import jax
import jax.numpy as jnp
from jax import lax
from jax.experimental import pallas as pl
from jax.experimental.pallas import tpu as pltpu

NUM_CODES = 1024
DIM = 64
TOK = 1024
N_STEPS = 16
N_TOK = N_STEPS * TOK
N_ELEM = N_TOK * DIM
COMMIT = 0.25
EPS = 1e-10


def _vq_body(x_ref, w_ref, qst_ref, idx_ref, scal_ref, wt_s, wb_s, w2_s,
             ci_s, cnt_s, sse_s):
    i = pl.program_id(0)

    @pl.when(i == 0)
    def _init():
        w = w_ref[...]
        wt_s[...] = w.T.astype(jnp.bfloat16)
        wb_s[...] = -2.0 * w.astype(jnp.bfloat16)
        w2_s[...] = jnp.sum(w * w, axis=1, keepdims=True)
        ci_s[...] = lax.broadcasted_iota(jnp.int32, (NUM_CODES, 1), 0
                                         ).astype(jnp.float32)
        cnt_s[...] = jnp.zeros_like(cnt_s)
        sse_s[...] = jnp.zeros_like(sse_s)

    xb = x_ref[0]
    x2 = jnp.sum(xb * xb, axis=0, keepdims=True)
    mmn = jnp.dot(wb_s[...], xb.astype(jnp.bfloat16),
                  preferred_element_type=jnp.float32)
    d = (x2 + w2_s[...]) + mmn
    dmin = jnp.min(d, axis=0, keepdims=True)
    cif = ci_s[...]
    idxf = jnp.min(jnp.where(d == dmin, cif, 2048.0),
                   axis=0, keepdims=True)
    idx_ref[0] = idxf.astype(jnp.int32)
    hitm = cif == idxf
    qt = jnp.dot(wt_s[...], hitm.astype(jnp.bfloat16),
                 preferred_element_type=jnp.float32)
    e = qt - xb
    qst_ref[0] = xb + e
    cnt_s[...] += jnp.sum(hitm.astype(jnp.float32),
                          axis=1, keepdims=True)
    sse_s[...] += jnp.sum(e * e, axis=(0, 1), keepdims=True)

    @pl.when(i == N_STEPS - 1)
    def _fin():
        loss = (1.0 + COMMIT) * sse_s[...] / N_ELEM
        p = cnt_s[...] * (1.0 / N_TOK)
        s = jnp.sum(p * jnp.log(p + EPS), axis=0, keepdims=True)
        perp = jnp.exp(-s)
        lane = lax.broadcasted_iota(jnp.int32, (1, 128), 1)
        scal_ref[...] = jnp.where(lane == 0,
                                  jnp.broadcast_to(loss, (1, 128)),
                                  jnp.broadcast_to(perp, (1, 128)))


def kernel(x, embedding_weight):
    B, C, H, W = x.shape
    xr = x.reshape(B, C, H * W)
    qst, idx, scal = pl.pallas_call(
        _vq_body,
        grid=(N_STEPS,),
        in_specs=[
            pl.BlockSpec((1, DIM, TOK), lambda i: (i, 0, 0)),
            pl.BlockSpec((NUM_CODES, DIM), lambda i: (0, 0)),
        ],
        out_specs=[
            pl.BlockSpec((1, DIM, TOK), lambda i: (i, 0, 0)),
            pl.BlockSpec((1, 1, TOK), lambda i: (i, 0, 0)),
            pl.BlockSpec((1, 128), lambda i: (0, 0)),
        ],
        out_shape=[
            jax.ShapeDtypeStruct((B, DIM, TOK), jnp.float32),
            jax.ShapeDtypeStruct((B, 1, TOK), jnp.int32),
            jax.ShapeDtypeStruct((1, 128), jnp.float32),
        ],
        scratch_shapes=[
            pltpu.VMEM((DIM, NUM_CODES), jnp.bfloat16),
            pltpu.VMEM((NUM_CODES, DIM), jnp.bfloat16),
            pltpu.VMEM((NUM_CODES, 1), jnp.float32),
            pltpu.VMEM((NUM_CODES, 1), jnp.float32),
            pltpu.VMEM((NUM_CODES, 1), jnp.float32),
            pltpu.VMEM((1, 1), jnp.float32),
        ],
    )(xr, embedding_weight)
    quantized_st = qst.reshape(B, C, H, W)
    idx_out = idx.reshape(B, H, W)
    loss = scal[0, 0]
    perplexity = scal[0, 1]
    return (quantized_st, loss, perplexity, idx_out)
